# baseline (speedup 1.0000x reference)
import jax
import jax.numpy as jnp
from jax import lax
from jax.experimental import pallas as pl
from jax.experimental.pallas import tpu as pltpu
from jax.experimental.pallas import tpu_sc as plsc

N = 10000
E = 320000
G = 16
CAT = 32
F = 64

NC = 2
NS = 16
NW = NC * NS
EPW = E // NW
CHUNK = 80
NCHUNK = EPW // CHUNK
NBUF = 5
NP = 10240
RPT = NP // NS

_SC_MESH = plsc.VectorSubcoreMesh(
    core_axis_name="c", subcore_axis_name="s", num_cores=NC, num_subcores=NS)


def _deg_body(dst_hbm, zeros_hbm, ones_hbm, out_hbm, dst_v, ones_v, deg_sh,
              dsem):
    cid = lax.axis_index("c")
    sid = lax.axis_index("s")
    wid = cid * NS + sid
    pltpu.sync_copy(dst_hbm.at[wid], dst_v)
    pltpu.sync_copy(ones_hbm, ones_v)
    pltpu.sync_copy(zeros_hbm.at[pl.ds(sid * RPT, RPT)],
                    deg_sh.at[pl.ds(sid * RPT, RPT)])
    plsc.subcore_barrier()

    def body(j, carry):
        pltpu.async_copy(ones_v, deg_sh.at[dst_v.at[j]], dsem, add=True)
        return carry

    lax.fori_loop(0, NCHUNK, body, 0)

    def drain(j, carry):
        pltpu.make_async_copy(ones_v, deg_sh.at[dst_v.at[0]], dsem).wait()
        return carry

    lax.fori_loop(0, NCHUNK, drain, 0)
    plsc.subcore_barrier()
    pltpu.sync_copy(deg_sh.at[pl.ds(sid * RPT, RPT)],
                    out_hbm.at[cid, pl.ds(sid * RPT, RPT)])


_SC_PARAMS = pltpu.CompilerParams(use_tc_tiling_on_sc=False)

_deg_kernel = pl.kernel(
    _deg_body,
    out_type=jax.ShapeDtypeStruct((NC, NP, 16), jnp.float32),
    mesh=_SC_MESH,
    compiler_params=_SC_PARAMS,
    scratch_types=[
        pltpu.VMEM((NCHUNK, CHUNK), jnp.int32),
        pltpu.VMEM((CHUNK, 16), jnp.float32),
        pltpu.VMEM_SHARED((NP, 16), jnp.float32),
        pltpu.SemaphoreType.DMA,
    ],
)


def _agg_body(y_hbm, src_hbm, dst_hbm, out_hbm, src_v, dst_v, rows_v, acc_sh,
              sems_g, sems_s):
    cid = lax.axis_index("c")
    sid = lax.axis_index("s")
    wid = cid * NS + sid
    pltpu.sync_copy(src_hbm.at[wid], src_v)
    pltpu.sync_copy(dst_hbm.at[wid], dst_v)
    for b in range(2):
        pltpu.async_copy(y_hbm.at[src_v.at[b]], rows_v.at[b], sems_g.at[b])
    pltpu.sync_copy(y_hbm.at[pl.ds(sid * RPT, RPT)],
                    acc_sh.at[pl.ds(sid * RPT, RPT)])
    plsc.subcore_barrier()

    def group(g, carry):
        for b in range(NBUF):
            jj = g * NBUF + b
            bn = (b + 2) % NBUF

            @pl.when(jj >= NBUF - 2)
            def _():
                pltpu.make_async_copy(
                    rows_v.at[bn], acc_sh.at[dst_v.at[0]], sems_s.at[bn]).wait()

            @pl.when(jj + 2 < NCHUNK)
            def _():
                pltpu.async_copy(
                    y_hbm.at[src_v.at[jj + 2]], rows_v.at[bn], sems_g.at[bn])

            pltpu.make_async_copy(
                y_hbm.at[src_v.at[jj]], rows_v.at[b], sems_g.at[b]).wait()
            pltpu.async_copy(
                rows_v.at[b], acc_sh.at[dst_v.at[jj]], sems_s.at[b], add=True)
        return carry

    lax.fori_loop(0, NCHUNK // NBUF, group, 0)
    for b in range(2, NBUF):
        pltpu.make_async_copy(
            rows_v.at[b], acc_sh.at[dst_v.at[0]], sems_s.at[b]).wait()
    plsc.subcore_barrier()
    pltpu.sync_copy(acc_sh.at[pl.ds(sid * RPT, RPT)],
                    out_hbm.at[cid, pl.ds(sid * RPT, RPT)])


_agg_kernel = pl.kernel(
    _agg_body,
    out_type=jax.ShapeDtypeStruct((NC, NP, F), jnp.float32),
    mesh=_SC_MESH,
    compiler_params=_SC_PARAMS,
    scratch_types=[
        pltpu.VMEM((NCHUNK, CHUNK), jnp.int32),
        pltpu.VMEM((NCHUNK, CHUNK), jnp.int32),
        pltpu.VMEM((NBUF, CHUNK, F), jnp.float32),
        pltpu.VMEM_SHARED((NP, F), jnp.float32),
        pltpu.SemaphoreType.DMA((NBUF,)),
        pltpu.SemaphoreType.DMA((NBUF,)),
    ],
)


def _mmT(a, w):
    return lax.dot_general(a, w, (((1,), (1,)), ((), ())),
                           preferred_element_type=jnp.float32)


def _dinv(deg_ref):
    deg = deg_ref[0, :N, 0:1] + deg_ref[1, :N, 0:1] + 1.0
    return lax.rsqrt(deg)


def _elu(h):
    neg = jnp.where(h > 0, 0.0, h)
    return jnp.where(h > 0, h, jnp.exp(neg) - 1.0)


def _front_body(x_ref, deg_ref, Wc_ref, bc_ref, Wi_ref, bi_ref, W1_ref,
                y_ref):
    dinv = _dinv(deg_ref)
    x = x_ref[...]
    s = jnp.maximum(_mmT(x[:, :CAT], Wc_ref[...]) + bc_ref[...], 0.0)
    h = (_mmT(x[:, CAT:], Wi_ref[...][:, :-CAT])
         + _mmT(s, Wi_ref[...][:, -CAT:]) + bi_ref[...])
    h = jnp.maximum(h, 0.0)
    y_ref[:N] = dinv * _mmT(h, W1_ref[...])
    y_ref[N:] = jnp.zeros((NP - N, F), jnp.float32)


_front = pl.pallas_call(
    _front_body,
    out_shape=jax.ShapeDtypeStruct((NP, F), jnp.float32),
)


def _mid_body(deg_ref, acc_ref, y_ref, b1_ref, W2_ref, out_ref):
    dinv = _dinv(deg_ref)
    agg = acc_ref[0, :N] + acc_ref[1, :N] - y_ref[:N]
    h = _elu(dinv * agg + b1_ref[...])
    out_ref[:N] = dinv * _mmT(h, W2_ref[...])
    out_ref[N:] = jnp.zeros((NP - N, F), jnp.float32)


_mid = pl.pallas_call(
    _mid_body,
    out_shape=jax.ShapeDtypeStruct((NP, F), jnp.float32),
)


def _back_body(deg_ref, acc_ref, y_ref, batch_ref, b2_ref, Wo_ref, bo_ref,
               out_ref):
    dinv = _dinv(deg_ref)
    agg = acc_ref[0, :N] + acc_ref[1, :N] - y_ref[:N]
    h = _elu(dinv * agg + b2_ref[...])
    onehot = (batch_ref[...] ==
              lax.broadcasted_iota(jnp.int32, (1, G), 1)).astype(jnp.float32)
    pooled = lax.dot_general(onehot, h, (((0,), (0,)), ((), ())),
                             preferred_element_type=jnp.float32)
    cnt = lax.dot_general(onehot, jnp.ones((N, 1), jnp.float32),
                          (((0,), (0,)), ((), ())),
                          preferred_element_type=jnp.float32)
    pooled = pooled / jnp.maximum(cnt, 1.0)
    out_ref[...] = _mmT(pooled, Wo_ref[...]) + bo_ref[...]


_back = pl.pallas_call(
    _back_body,
    out_shape=jax.ShapeDtypeStruct((G, F), jnp.float32),
)


def kernel(x, edge_index, batch, Wc, bc, Wi, bi, W1, b1, W2, b2, Wo, bo):
    ei = edge_index.astype(jnp.int32)
    src3 = ei[0].reshape(NW, NCHUNK, CHUNK)
    dst3 = ei[1].reshape(NW, NCHUNK, CHUNK)
    zeros16 = jnp.zeros((NP, 16), jnp.float32)
    ones16 = jnp.ones((CHUNK, 16), jnp.float32)

    degp = _deg_kernel(dst3, zeros16, ones16)
    y1 = _front(x, degp, Wc, bc.reshape(1, -1), Wi, bi.reshape(1, -1), W1)
    acc1 = _agg_kernel(y1, src3, dst3)
    y2 = _mid(degp, acc1, y1, b1.reshape(1, -1), W2)
    acc2 = _agg_kernel(y2, src3, dst3)
    return _back(degp, acc2, y2, batch.astype(jnp.int32).reshape(N, 1),
                 b2.reshape(1, -1), Wo, bo.reshape(1, -1))


# speedup vs baseline: 1.0533x; 1.0533x over previous
import jax
import jax.numpy as jnp
from jax import lax
from jax.experimental import pallas as pl
from jax.experimental.pallas import tpu as pltpu
from jax.experimental.pallas import tpu_sc as plsc

N = 10000
E = 320000
G = 16
CAT = 32
F = 64

NC = 2
NS = 16
NW = NC * NS
EPW = E // NW
CHUNK = 80
NCHUNK = EPW // CHUNK
NBUF = 5
NP = 10240
RPT = NP // NS

_SC_MESH = plsc.VectorSubcoreMesh(
    core_axis_name="c", subcore_axis_name="s", num_cores=NC, num_subcores=NS)
_SC_PARAMS = pltpu.CompilerParams(use_tc_tiling_on_sc=False)


def _stage_dst_rows(dst_hbm, dst_v, isem, base):
    def fire(j, carry):
        off = pl.multiple_of(base + j * CHUNK, 8)
        pltpu.async_copy(dst_hbm.at[pl.ds(off, CHUNK)], dst_v.at[j], isem)
        return carry

    lax.fori_loop(0, NCHUNK, fire, 0)


def _drain_dst_rows(dst_hbm, dst_v, isem):
    def drain(j, carry):
        pltpu.make_async_copy(
            dst_hbm.at[pl.ds(0, CHUNK)], dst_v.at[0], isem).wait()
        return carry

    lax.fori_loop(0, NCHUNK, drain, 0)


def _deg_body(dst_hbm, zeros_hbm, ones_hbm, out_hbm, dst_v, ones_v, deg_sh,
              isem, dsem):
    cid = lax.axis_index("c")
    sid = lax.axis_index("s")
    wid = cid * NS + sid
    _stage_dst_rows(dst_hbm, dst_v, isem, wid * EPW)
    pltpu.sync_copy(ones_hbm, ones_v)
    pltpu.sync_copy(zeros_hbm.at[pl.ds(sid * RPT, RPT)],
                    deg_sh.at[pl.ds(sid * RPT, RPT)])
    _drain_dst_rows(dst_hbm, dst_v, isem)
    plsc.subcore_barrier()

    def body(j, carry):
        pltpu.async_copy(ones_v, deg_sh.at[dst_v.at[j]], dsem, add=True)
        return carry

    lax.fori_loop(0, NCHUNK, body, 0)

    def drain(j, carry):
        pltpu.make_async_copy(ones_v, deg_sh.at[dst_v.at[0]], dsem).wait()
        return carry

    lax.fori_loop(0, NCHUNK, drain, 0)
    plsc.subcore_barrier()
    pltpu.sync_copy(deg_sh.at[pl.ds(sid * RPT, RPT)],
                    out_hbm.at[cid, pl.ds(sid * RPT, RPT)])


_deg_kernel = pl.kernel(
    _deg_body,
    out_type=jax.ShapeDtypeStruct((NC, NP, 16), jnp.float32),
    mesh=_SC_MESH,
    compiler_params=_SC_PARAMS,
    scratch_types=[
        pltpu.VMEM((NCHUNK, CHUNK), jnp.int32),
        pltpu.VMEM((CHUNK, 16), jnp.float32),
        pltpu.VMEM_SHARED((NP, 16), jnp.float32),
        pltpu.SemaphoreType.DMA,
        pltpu.SemaphoreType.DMA,
    ],
)


def _agg_body(y_hbm, src_hbm, dst_hbm, out_hbm, src_v, dst_v, rows_v, acc_sh,
              isem, sems):
    cid = lax.axis_index("c")
    sid = lax.axis_index("s")
    wid = cid * NS + sid
    base = wid * EPW
    pltpu.sync_copy(src_hbm.at[pl.ds(base, EPW)], src_v)
    _stage_dst_rows(dst_hbm, dst_v, isem, base)
    for b in range(NBUF):
        pltpu.async_copy(
            y_hbm.at[src_v.at[pl.ds(b * CHUNK, CHUNK)]], rows_v.at[b],
            sems.at[b])
    pltpu.sync_copy(y_hbm.at[pl.ds(sid * RPT, RPT)],
                    acc_sh.at[pl.ds(sid * RPT, RPT)])
    _drain_dst_rows(dst_hbm, dst_v, isem)
    plsc.subcore_barrier()

    def group(g, carry):
        for b in range(NBUF):
            jj = g * NBUF + b
            off = pl.multiple_of(jj * CHUNK, 8)
            pltpu.make_async_copy(
                y_hbm.at[src_v.at[pl.ds(off, CHUNK)]], rows_v.at[b],
                sems.at[b]).wait()
            pltpu.sync_copy(rows_v.at[b], acc_sh.at[dst_v.at[jj]], add=True)
            nxt = jj + NBUF

            @pl.when(nxt < NCHUNK)
            def _():
                noff = pl.multiple_of(nxt * CHUNK, 8)
                pltpu.async_copy(
                    y_hbm.at[src_v.at[pl.ds(noff, CHUNK)]], rows_v.at[b],
                    sems.at[b])
        return carry

    lax.fori_loop(0, NCHUNK // NBUF, group, 0)
    plsc.subcore_barrier()
    pltpu.sync_copy(acc_sh.at[pl.ds(sid * RPT, RPT)],
                    out_hbm.at[cid, pl.ds(sid * RPT, RPT)])


_agg_kernel = pl.kernel(
    _agg_body,
    out_type=jax.ShapeDtypeStruct((NC, NP, F), jnp.float32),
    mesh=_SC_MESH,
    compiler_params=_SC_PARAMS,
    scratch_types=[
        pltpu.VMEM((EPW,), jnp.int32),
        pltpu.VMEM((NCHUNK, CHUNK), jnp.int32),
        pltpu.VMEM((NBUF, CHUNK, F), jnp.float32),
        pltpu.VMEM_SHARED((NP, F), jnp.float32),
        pltpu.SemaphoreType.DMA,
        pltpu.SemaphoreType.DMA((NBUF,)),
    ],
)


def _mmT(a, w):
    return lax.dot_general(a, w, (((1,), (1,)), ((), ())),
                           preferred_element_type=jnp.float32)


def _dinv(deg_ref):
    deg = deg_ref[0, :N] + deg_ref[1, :N] + 1.0
    return lax.rsqrt(deg)


def _elu(h):
    neg = jnp.where(h > 0, 0.0, h)
    return jnp.where(h > 0, h, jnp.exp(neg) - 1.0)


def _front_body(x_ref, deg_ref, Wc_ref, bc_ref, Wi_ref, bi_ref, W1_ref,
                y_ref):
    dinv = _dinv(deg_ref)
    x = x_ref[...]
    s = jnp.maximum(_mmT(x[:, :CAT], Wc_ref[...]) + bc_ref[...], 0.0)
    h = (_mmT(x[:, CAT:], Wi_ref[...][:, :-CAT])
         + _mmT(s, Wi_ref[...][:, -CAT:]) + bi_ref[...])
    h = jnp.maximum(h, 0.0)
    y_ref[:N] = dinv * _mmT(h, W1_ref[...])
    y_ref[N:] = jnp.zeros((NP - N, F), jnp.float32)


_front = pl.pallas_call(
    _front_body,
    out_shape=jax.ShapeDtypeStruct((NP, F), jnp.float32),
)


def _mid_body(deg_ref, acc_ref, y_ref, b1_ref, W2_ref, out_ref):
    dinv = _dinv(deg_ref)
    agg = acc_ref[0, :N] + acc_ref[1, :N] - y_ref[:N]
    h = _elu(dinv * agg + b1_ref[...])
    out_ref[:N] = dinv * _mmT(h, W2_ref[...])
    out_ref[N:] = jnp.zeros((NP - N, F), jnp.float32)


_mid = pl.pallas_call(
    _mid_body,
    out_shape=jax.ShapeDtypeStruct((NP, F), jnp.float32),
)


def _back_body(deg_ref, acc_ref, y_ref, batch_ref, b2_ref, Wo_ref, bo_ref,
               out_ref):
    dinv = _dinv(deg_ref)
    agg = acc_ref[0, :N] + acc_ref[1, :N] - y_ref[:N]
    h = _elu(dinv * agg + b2_ref[...])
    onehot = (batch_ref[...] ==
              lax.broadcasted_iota(jnp.int32, (1, G), 1)).astype(jnp.float32)
    pooled = lax.dot_general(onehot, h, (((0,), (0,)), ((), ())),
                             preferred_element_type=jnp.float32)
    cnt = lax.dot_general(onehot, jnp.ones((N, 1), jnp.float32),
                          (((0,), (0,)), ((), ())),
                          preferred_element_type=jnp.float32)
    pooled = pooled / jnp.maximum(cnt, 1.0)
    out_ref[...] = _mmT(pooled, Wo_ref[...]) + bo_ref[...]


_back = pl.pallas_call(
    _back_body,
    out_shape=jax.ShapeDtypeStruct((G, F), jnp.float32),
)


def kernel(x, edge_index, batch, Wc, bc, Wi, bi, W1, b1, W2, b2, Wo, bo):
    ei = edge_index.astype(jnp.int32)
    src1 = ei[0]
    dst1 = ei[1]
    zeros16 = jnp.zeros((NP, 16), jnp.float32)
    ones16 = jnp.ones((CHUNK, 16), jnp.float32)

    degp = _deg_kernel(dst1, zeros16, ones16)
    degs = degp[:, :, 0:1]
    y1 = _front(x, degs, Wc, bc.reshape(1, -1), Wi, bi.reshape(1, -1), W1)
    acc1 = _agg_kernel(y1, src1, dst1)
    y2 = _mid(degs, acc1, y1, b1.reshape(1, -1), W2)
    acc2 = _agg_kernel(y2, src1, dst1)
    return _back(degs, acc2, y2, batch.astype(jnp.int32).reshape(N, 1),
                 b2.reshape(1, -1), Wo, bo.reshape(1, -1))


# speedup vs baseline: 1.0977x; 1.0421x over previous
import jax
import jax.numpy as jnp
from jax import lax
from jax.experimental import pallas as pl
from jax.experimental.pallas import tpu as pltpu
from jax.experimental.pallas import tpu_sc as plsc

N = 10000
E = 320000
G = 16
CAT = 32
F = 64

NC = 2
NS = 16
NW = NC * NS
EPW = E // NW
CHUNK = 80
NCHUNK = EPW // CHUNK
NBUF = 5
NP = 10240
RPT = NP // NS
GB = 1024

_SC_MESH = plsc.VectorSubcoreMesh(
    core_axis_name="c", subcore_axis_name="s", num_cores=NC, num_subcores=NS)
_SC_PARAMS = pltpu.CompilerParams(use_tc_tiling_on_sc=False,
                                  needs_layout_passes=False)


def _stage_dst_rows(dst_hbm, dst_v, isem, base):
    def fire(j, carry):
        off = pl.multiple_of(base + j * CHUNK, 8)
        pltpu.async_copy(dst_hbm.at[pl.ds(off, CHUNK)], dst_v.at[j], isem)
        return carry

    lax.fori_loop(0, NCHUNK, fire, 0)


def _drain_dst_rows(dst_hbm, dst_v, isem):
    def drain(j, carry):
        pltpu.make_async_copy(
            dst_hbm.at[pl.ds(0, CHUNK)], dst_v.at[0], isem).wait()
        return carry

    lax.fori_loop(0, NCHUNK, drain, 0)


def _deg_body(dst_hbm, zeros_hbm, ones_hbm, out_hbm, dst_v, ones_v, deg_sh,
              degrow_v, cmp_v, isem, dsem):
    cid = lax.axis_index("c")
    sid = lax.axis_index("s")
    wid = cid * NS + sid
    _stage_dst_rows(dst_hbm, dst_v, isem, wid * EPW)
    pltpu.sync_copy(ones_hbm, ones_v)
    pltpu.sync_copy(zeros_hbm.at[pl.ds(sid * RPT, RPT)],
                    deg_sh.at[pl.ds(sid * RPT, RPT)])
    _drain_dst_rows(dst_hbm, dst_v, isem)
    plsc.subcore_barrier()

    def body(j, carry):
        pltpu.async_copy(ones_v, deg_sh.at[dst_v.at[j]], dsem, add=True)
        return carry

    lax.fori_loop(0, NCHUNK, body, 0)

    def drain(j, carry):
        pltpu.make_async_copy(ones_v, deg_sh.at[dst_v.at[0]], dsem).wait()
        return carry

    lax.fori_loop(0, NCHUNK, drain, 0)
    plsc.subcore_barrier()
    pltpu.sync_copy(deg_sh.at[pl.ds(sid * RPT, RPT)], degrow_v)
    col0 = jnp.zeros((16,), jnp.int32)
    lanes = lax.iota(jnp.int32, 16)

    def compact(k, carry):
        rows = k * 16 + lanes
        vals = plsc.load_gather(degrow_v, [rows, col0])
        cmp_v[pl.ds(pl.multiple_of(k * 16, 8), 16)] = vals
        return carry

    lax.fori_loop(0, RPT // 16, compact, 0)
    pltpu.sync_copy(cmp_v, out_hbm.at[cid, 0, pl.ds(sid * RPT, RPT)])


_deg_kernel = pl.kernel(
    _deg_body,
    out_type=jax.ShapeDtypeStruct((NC, 1, NP), jnp.float32),
    mesh=_SC_MESH,
    compiler_params=_SC_PARAMS,
    scratch_types=[
        pltpu.VMEM((NCHUNK, CHUNK), jnp.int32),
        pltpu.VMEM((CHUNK, 16), jnp.float32),
        pltpu.VMEM_SHARED((NP, 16), jnp.float32),
        pltpu.VMEM((RPT, 16), jnp.float32),
        pltpu.VMEM((RPT,), jnp.float32),
        pltpu.SemaphoreType.DMA,
        pltpu.SemaphoreType.DMA,
    ],
)


def _agg_body(y_hbm, src_hbm, dst_hbm, out_hbm, src_v, dst_v, rows_v, acc_sh,
              isem, sems):
    cid = lax.axis_index("c")
    sid = lax.axis_index("s")
    wid = cid * NS + sid
    base = wid * EPW
    pltpu.sync_copy(src_hbm.at[pl.ds(base, EPW)], src_v)
    _stage_dst_rows(dst_hbm, dst_v, isem, base)
    for b in range(NBUF):
        pltpu.async_copy(
            y_hbm.at[src_v.at[pl.ds(b * CHUNK, CHUNK)]], rows_v.at[b],
            sems.at[b])
    pltpu.sync_copy(y_hbm.at[pl.ds(sid * RPT, RPT)],
                    acc_sh.at[pl.ds(sid * RPT, RPT)])
    _drain_dst_rows(dst_hbm, dst_v, isem)
    plsc.subcore_barrier()

    def group(g, carry):
        for b in range(NBUF):
            jj = g * NBUF + b
            off = pl.multiple_of(jj * CHUNK, 8)
            pltpu.make_async_copy(
                y_hbm.at[src_v.at[pl.ds(off, CHUNK)]], rows_v.at[b],
                sems.at[b]).wait()
            pltpu.sync_copy(rows_v.at[b], acc_sh.at[dst_v.at[jj]], add=True)
            nxt = jj + NBUF

            @pl.when(nxt < NCHUNK)
            def _():
                noff = pl.multiple_of(nxt * CHUNK, 8)
                pltpu.async_copy(
                    y_hbm.at[src_v.at[pl.ds(noff, CHUNK)]], rows_v.at[b],
                    sems.at[b])
        return carry

    lax.fori_loop(0, NCHUNK // NBUF, group, 0)
    plsc.subcore_barrier()
    pltpu.sync_copy(acc_sh.at[pl.ds(sid * RPT, RPT)],
                    out_hbm.at[cid, pl.ds(sid * RPT, RPT)])


_agg_kernel = pl.kernel(
    _agg_body,
    out_type=jax.ShapeDtypeStruct((NC, NP, F), jnp.float32),
    mesh=_SC_MESH,
    compiler_params=_SC_PARAMS,
    scratch_types=[
        pltpu.VMEM((EPW,), jnp.int32),
        pltpu.VMEM((NCHUNK, CHUNK), jnp.int32),
        pltpu.VMEM((NBUF, CHUNK, F), jnp.float32),
        pltpu.VMEM_SHARED((NP, F), jnp.float32),
        pltpu.SemaphoreType.DMA,
        pltpu.SemaphoreType.DMA((NBUF,)),
    ],
)


def _mmT(a, w):
    return lax.dot_general(a, w, (((1,), (1,)), ((), ())),
                           preferred_element_type=jnp.float32)


def _dinv_col(deg_ref, rows):
    deg = deg_ref[0, 0:1, :rows] + deg_ref[1, 0:1, :rows] + 1.0
    return lax.transpose(lax.rsqrt(deg), (1, 0))


def _elu(h):
    neg = jnp.where(h > 0, 0.0, h)
    return jnp.where(h > 0, h, jnp.exp(neg) - 1.0)


def _front_body(x_ref, deg_ref, Wc_ref, bc_ref, Wi_ref, bi_ref, W1_ref,
                y_ref):
    dinv = _dinv_col(deg_ref, N)
    x = x_ref[...]
    s = jnp.maximum(_mmT(x[:, :CAT], Wc_ref[...]) + bc_ref[...], 0.0)
    h = (_mmT(x[:, CAT:], Wi_ref[...][:, :-CAT])
         + _mmT(s, Wi_ref[...][:, -CAT:]) + bi_ref[...])
    h = jnp.maximum(h, 0.0)
    y_ref[:N] = dinv * _mmT(h, W1_ref[...])
    y_ref[N:] = jnp.zeros((NP - N, F), jnp.float32)


_front = pl.pallas_call(
    _front_body,
    out_shape=jax.ShapeDtypeStruct((NP, F), jnp.float32),
)


def _mid_body(deg_ref, acc_ref, y_ref, b1_ref, W2_ref, out_ref):
    i = pl.program_id(0)
    dinv = _dinv_col(deg_ref, GB)
    agg = acc_ref[0] + acc_ref[1] - y_ref[...]
    h = _elu(dinv * agg + b1_ref[...])
    val = dinv * _mmT(h, W2_ref[...])
    rid = i * GB + lax.broadcasted_iota(jnp.int32, (GB, 1), 0)
    out_ref[...] = jnp.where(rid < N, val, 0.0)


_mid = pl.pallas_call(
    _mid_body,
    grid=(NP // GB,),
    in_specs=[
        pl.BlockSpec((2, 1, GB), lambda i: (0, 0, i)),
        pl.BlockSpec((2, GB, F), lambda i: (0, i, 0)),
        pl.BlockSpec((GB, F), lambda i: (i, 0)),
        pl.BlockSpec((1, F), lambda i: (0, 0)),
        pl.BlockSpec((F, F), lambda i: (0, 0)),
    ],
    out_specs=pl.BlockSpec((GB, F), lambda i: (i, 0)),
    out_shape=jax.ShapeDtypeStruct((NP, F), jnp.float32),
)


def _back_body(deg_ref, acc_ref, y_ref, batch_ref, b2_ref, Wo_ref, bo_ref,
               out_ref):
    dinv = _dinv_col(deg_ref, N)
    agg = acc_ref[0, :N] + acc_ref[1, :N] - y_ref[:N]
    h = _elu(dinv * agg + b2_ref[...])
    onehot = (batch_ref[...] ==
              lax.broadcasted_iota(jnp.int32, (1, G), 1)).astype(jnp.float32)
    pooled = lax.dot_general(onehot, h, (((0,), (0,)), ((), ())),
                             preferred_element_type=jnp.float32)
    cnt = lax.dot_general(onehot, jnp.ones((N, 1), jnp.float32),
                          (((0,), (0,)), ((), ())),
                          preferred_element_type=jnp.float32)
    pooled = pooled / jnp.maximum(cnt, 1.0)
    out_ref[...] = _mmT(pooled, Wo_ref[...]) + bo_ref[...]


_back = pl.pallas_call(
    _back_body,
    out_shape=jax.ShapeDtypeStruct((G, F), jnp.float32),
)


def kernel(x, edge_index, batch, Wc, bc, Wi, bi, W1, b1, W2, b2, Wo, bo):
    ei = edge_index.astype(jnp.int32)
    zeros16 = jnp.zeros((NP, 16), jnp.float32)
    ones16 = jnp.ones((CHUNK, 16), jnp.float32)

    src1 = ei[0]
    dst1 = ei[1]
    degs = _deg_kernel(dst1, zeros16, ones16)
    y1 = _front(x, degs, Wc, bc.reshape(1, -1), Wi, bi.reshape(1, -1), W1)
    acc1 = _agg_kernel(y1, src1, dst1)
    y2 = _mid(degs, acc1, y1, b1.reshape(1, -1), W2)
    acc2 = _agg_kernel(y2, src1, dst1)
    return _back(degs, acc2, y2, batch.astype(jnp.int32).reshape(N, 1),
                 b2.reshape(1, -1), Wo, bo.reshape(1, -1))
